# Initial kernel scaffold; baseline (speedup 1.0000x reference)
import jax
import jax.numpy as jnp
from jax.experimental import pallas as pl


def kernel(x, edge_index, W_in, b_in, conv_w, env_Wlocal, env_mlp_w1, env_mlp_b1, env_mlp_w2, env_mlp_b2, env_alpha, env_fc_w, env_fc_b, W_out, b_out):
    raise NotImplementedError("write your pallas kernel here")


# speedup vs baseline: 8.5163x; 8.5163x over previous
import functools

import jax
import jax.numpy as jnp
from jax import lax
from jax.experimental import pallas as pl
from jax.experimental.pallas import tpu as pltpu
from jax.experimental.pallas import tpu_sc as plsc

N = 10000
F = 128
K = 3
L = 2

NPAD = 10240
BN = 1024
NBLK = NPAD // BN
CH = 128
NW = 32
ROWS_PER_TILE = NPAD // 16

def _mesh():
    return plsc.VectorSubcoreMesh(core_axis_name="c", subcore_axis_name="s")


def _sc_degree(epad):
    epw = epad // NW

    @functools.partial(
        pl.kernel,
        mesh=_mesh(),
        out_type=jax.ShapeDtypeStruct((NW, NPAD), jnp.float32),
        scratch_types=[
            pltpu.VMEM((epw,), jnp.int32),
            pltpu.VMEM((NPAD,), jnp.float32),
        ],
        compiler_params=pltpu.CompilerParams(use_tc_tiling_on_sc=False, needs_layout_passes=False),
    )
    def deg(col_hbm, out_hbm, colv, dpart):
        c = lax.axis_index("c")
        s = lax.axis_index("s")
        wid = c * 16 + s
        pltpu.sync_copy(col_hbm.at[pl.ds(wid * epw, epw)], colv)

        def zero(j, _):
            dpart[pl.ds(j * 16, 16)] = jnp.zeros((16,), jnp.float32)
            return 0

        lax.fori_loop(0, NPAD // 16, zero, 0)
        ones = jnp.ones((16,), jnp.float32)

        def body(j, _):
            idx = colv[pl.ds(j * 16, 16)]
            plsc.addupdate_scatter(dpart, [idx], ones)
            return 0

        lax.fori_loop(0, epw // 16, body, 0)
        pltpu.sync_copy(dpart, out_hbm.at[wid])

    return deg


def _sc_scatter(epad):
    epw = epad // NW
    nch = epw // CH

    @functools.partial(
        pl.kernel,
        mesh=_mesh(),
        out_type=jax.ShapeDtypeStruct((2, NPAD, F), jnp.float32),
        scratch_types=[
            pltpu.VMEM((CH,), jnp.int32),
            pltpu.VMEM((CH,), jnp.int32),
            pltpu.VMEM((CH, F), jnp.float32),
            pltpu.VMEM_SHARED((NPAD, F), jnp.float32),
            pltpu.SemaphoreType.DMA,
        ],
    )
    def scat(hs_hbm, row_hbm, col_hbm, zf_hbm, out_hbm, ridx, cidx, rows, acc, sem):
        c = lax.axis_index("c")
        s = lax.axis_index("s")
        wid = c * 16 + s
        rpt = ROWS_PER_TILE
        pltpu.sync_copy(zf_hbm.at[pl.ds(s * rpt, rpt)], acc.at[pl.ds(s * rpt, rpt)])
        plsc.subcore_barrier()

        wstart = wid * epw

        def body(i, _):
            base = wstart + i * CH
            pltpu.sync_copy(row_hbm.at[pl.ds(base, CH)], ridx)
            pltpu.sync_copy(col_hbm.at[pl.ds(base, CH)], cidx)
            pltpu.async_copy(hs_hbm.at[ridx], rows, sem).wait()
            pltpu.sync_copy(rows, acc.at[cidx], add=True)
            return 0

        lax.fori_loop(0, nch, body, 0)
        plsc.subcore_barrier()
        pltpu.sync_copy(acc.at[pl.ds(s * rpt, rpt)],
                        out_hbm.at[c, pl.ds(s * rpt, rpt)])

    return scat


def _dn_from_parts(d):
    dn = jnp.where(d > 0.0, lax.rsqrt(jnp.maximum(d, 1e-30)), 0.0)
    return dn[:, 0:1]


def _pre_body(x_ref, w_ref, b_ref, dp_ref, h_ref, hs_ref, gs_ref):
    pid = pl.program_id(0)
    h = jnp.maximum(jnp.dot(x_ref[...], w_ref[...],
                            preferred_element_type=jnp.float32) + b_ref[...], 0.0)
    r = pid * BN + lax.broadcasted_iota(jnp.int32, (BN, 1), 0)
    h = jnp.where(r < N, h, 0.0)
    dn = _dn_from_parts(dp_ref[...])
    h_ref[...] = h
    hs_ref[...] = h * dn

    @pl.when(pid == 0)
    def _():
        gs_ref[...] = jnp.zeros_like(gs_ref)

    gs_ref[...] += jnp.sum(h, axis=0, keepdims=True)


def _tc_pre(xp, w_in, b_in, dparts):
    return pl.pallas_call(
        _pre_body,
        grid=(NBLK,),
        in_specs=[
            pl.BlockSpec((BN, F), lambda b: (b, 0)),
            pl.BlockSpec((F, F), lambda b: (0, 0)),
            pl.BlockSpec((1, F), lambda b: (0, 0)),
            pl.BlockSpec((BN, 16), lambda b: (b, 0)),
        ],
        out_specs=[
            pl.BlockSpec((BN, F), lambda b: (b, 0)),
            pl.BlockSpec((BN, F), lambda b: (b, 0)),
            pl.BlockSpec((1, F), lambda b: (0, 0)),
        ],
        out_shape=[
            jax.ShapeDtypeStruct((NPAD, F), jnp.float32),
            jax.ShapeDtypeStruct((NPAD, F), jnp.float32),
            jax.ShapeDtypeStruct((1, F), jnp.float32),
        ],
    )(xp, w_in, b_in, dparts)


def _mix_body(last, acc_ref, dp_ref, h_ref, gs_ref, wl_ref, w1_ref, b1_ref,
              w2_ref, b2_ref, al_ref, fcw_ref, fcb_ref, cw_ref, wo_ref, bo_ref,
              *out_refs):
    pid = pl.program_id(0)
    dn = _dn_from_parts(dp_ref[...])
    acc = acc_ref[...]
    hi = dn * (acc[0] + acc[1])
    h = h_ref[...]

    gp = gs_ref[...] * (1.0 / N)
    ge = jnp.maximum(jnp.dot(gp, w1_ref[...], preferred_element_type=jnp.float32)
                     + b1_ref[...], 0.0)
    ge = jnp.dot(ge, w2_ref[...], preferred_element_type=jnp.float32) + b2_ref[...]
    wgt = jax.nn.sigmoid(al_ref[...])

    local = jnp.dot(hi, wl_ref[...], preferred_element_type=jnp.float32)
    comb = wgt * local + (1.0 - wgt) * ge
    logits = jnp.dot(comb, fcw_ref[...], preferred_element_type=jnp.float32) + fcb_ref[...]
    m = jnp.max(logits, axis=1, keepdims=True)
    ee = jnp.exp(logits - m)
    e = ee / jnp.sum(ee, axis=1, keepdims=True)

    cw = cw_ref[...]
    out = h
    for k in range(K):
        ok = (jnp.dot(hi, cw[k, :F, :], preferred_element_type=jnp.float32)
              + jnp.dot(h, cw[k, F:, :], preferred_element_type=jnp.float32))
        out = out + e[:, k:k + 1] * ok
    hn = jnp.maximum(out, 0.0)

    if last:
        y_ref, = out_refs
        y_ref[...] = jnp.dot(hn, wo_ref[...], preferred_element_type=jnp.float32) + bo_ref[...]
    else:
        h_out, hs_out, gs_out = out_refs
        h_out[...] = hn
        hs_out[...] = hn * dn

        @pl.when(pid == 0)
        def _():
            gs_out[...] = jnp.zeros_like(gs_out)

        gs_out[...] += jnp.sum(hn, axis=0, keepdims=True)


def _tc_mix(last, accs, dparts, h, gsum, wl, w1, b1, w2, b2, al, fcw, fcb, cw, wo, bo):
    if last:
        out_specs = [pl.BlockSpec((BN, F), lambda b: (b, 0))]
        out_shape = [jax.ShapeDtypeStruct((NPAD, F), jnp.float32)]
    else:
        out_specs = [
            pl.BlockSpec((BN, F), lambda b: (b, 0)),
            pl.BlockSpec((BN, F), lambda b: (b, 0)),
            pl.BlockSpec((1, F), lambda b: (0, 0)),
        ]
        out_shape = [
            jax.ShapeDtypeStruct((NPAD, F), jnp.float32),
            jax.ShapeDtypeStruct((NPAD, F), jnp.float32),
            jax.ShapeDtypeStruct((1, F), jnp.float32),
        ]
    return pl.pallas_call(
        functools.partial(_mix_body, last),
        grid=(NBLK,),
        in_specs=[
            pl.BlockSpec((2, BN, F), lambda b: (0, b, 0)),
            pl.BlockSpec((BN, 16), lambda b: (b, 0)),
            pl.BlockSpec((BN, F), lambda b: (b, 0)),
            pl.BlockSpec((1, F), lambda b: (0, 0)),
            pl.BlockSpec((F, F), lambda b: (0, 0)),
            pl.BlockSpec((F, F), lambda b: (0, 0)),
            pl.BlockSpec((1, F), lambda b: (0, 0)),
            pl.BlockSpec((F, F), lambda b: (0, 0)),
            pl.BlockSpec((1, F), lambda b: (0, 0)),
            pl.BlockSpec((1, F), lambda b: (0, 0)),
            pl.BlockSpec((F, F), lambda b: (0, 0)),
            pl.BlockSpec((1, F), lambda b: (0, 0)),
            pl.BlockSpec((K, 2 * F, F), lambda b: (0, 0, 0)),
            pl.BlockSpec((F, F), lambda b: (0, 0)),
            pl.BlockSpec((1, F), lambda b: (0, 0)),
        ],
        out_specs=out_specs,
        out_shape=out_shape,
    )(accs, dparts, h, gsum, wl, w1, b1, w2, b2, al, fcw, fcb, cw, wo, bo)


def kernel(x, edge_index, W_in, b_in, conv_w, env_Wlocal, env_mlp_w1, env_mlp_b1,
           env_mlp_w2, env_mlp_b2, env_alpha, env_fc_w, env_fc_b, W_out, b_out):
    ei = edge_index.astype(jnp.int32)
    E = ei.shape[1]
    epad = ((E + NW * CH - 1) // (NW * CH)) * (NW * CH)
    pad = epad - E
    rowp = jnp.concatenate([ei[0], jnp.full((pad,), N, jnp.int32)])
    colp = jnp.concatenate([ei[1], jnp.full((pad,), N, jnp.int32)])

    xp = jnp.zeros((NPAD, F), jnp.float32).at[:N].set(x)
    zf = jnp.zeros((NPAD, F), jnp.float32)

    dparts = _sc_degree(epad)(colp)
    d16 = jnp.broadcast_to(jnp.sum(dparts, axis=0)[:, None], (NPAD, 16))

    b_in2 = b_in.reshape(1, F)
    h, hs, gsum = _tc_pre(xp, W_in, b_in2, d16)

    fcw_p = jnp.zeros((L, F, F), jnp.float32).at[:, :, :K].set(env_fc_w)
    fcb_p = jnp.full((L, 1, F), -1e30, jnp.float32).at[:, 0, :K].set(env_fc_b)
    bo2 = b_out.reshape(1, F)

    scat = _sc_scatter(epad)
    for l in range(L):
        accs = scat(hs, rowp, colp, zf)
        al = jnp.full((1, F), env_alpha[l])
        outs = _tc_mix(l == L - 1, accs, d16, h, gsum,
                       env_Wlocal[l], env_mlp_w1[l], env_mlp_b1[l].reshape(1, F),
                       env_mlp_w2[l], env_mlp_b2[l].reshape(1, F), al,
                       fcw_p[l], fcb_p[l], conv_w[l], W_out, bo2)
        if l == L - 1:
            y, = outs
        else:
            h, hs, gsum = outs
    return y[:N]


# speedup vs baseline: 13.5508x; 1.5912x over previous
import functools

import jax
import jax.numpy as jnp
from jax import lax
from jax.experimental import pallas as pl
from jax.experimental.pallas import tpu as pltpu
from jax.experimental.pallas import tpu_sc as plsc

N = 10000
F = 128
K = 3
L = 2

NPAD = 10240
BN = 1024
NBLK = NPAD // BN
CH = 128
FH = F // 2
NW = 32
ROWS_PER_TILE = NPAD // 16

def _mesh():
    return plsc.VectorSubcoreMesh(core_axis_name="c", subcore_axis_name="s")


def _sc_degree(epad):
    epw = epad // NW

    @functools.partial(
        pl.kernel,
        mesh=_mesh(),
        out_type=jax.ShapeDtypeStruct((NW, NPAD), jnp.float32),
        scratch_types=[
            pltpu.VMEM((epw,), jnp.int32),
            pltpu.VMEM((NPAD,), jnp.float32),
        ],
        compiler_params=pltpu.CompilerParams(use_tc_tiling_on_sc=False, needs_layout_passes=False),
    )
    def deg(col_hbm, out_hbm, colv, dpart):
        c = lax.axis_index("c")
        s = lax.axis_index("s")
        wid = c * 16 + s
        pltpu.sync_copy(col_hbm.at[pl.ds(wid * epw, epw)], colv)

        def zero(j, _):
            dpart[pl.ds(j * 16, 16)] = jnp.zeros((16,), jnp.float32)
            return 0

        lax.fori_loop(0, NPAD // 16, zero, 0)
        ones = jnp.ones((16,), jnp.float32)

        def body(j, _):
            idx = colv[pl.ds(j * 16, 16)]
            plsc.addupdate_scatter(dpart, [idx], ones)
            return 0

        lax.fori_loop(0, epw // 16, body, 0)
        pltpu.sync_copy(dpart, out_hbm.at[wid])

    return deg


def _sc_scatter(epad):
    ept = epad // 16
    nch = ept // CH
    assert nch % 2 == 0

    @functools.partial(
        pl.kernel,
        mesh=_mesh(),
        out_type=jax.ShapeDtypeStruct((2, NPAD, FH), jnp.float32),
        scratch_types=[
            pltpu.VMEM((nch, CH), jnp.int32),
            pltpu.VMEM((nch, CH), jnp.int32),
            pltpu.VMEM((CH, FH), jnp.float32),
            pltpu.VMEM((CH, FH), jnp.float32),
            pltpu.VMEM_SHARED((NPAD, FH), jnp.float32),
            pltpu.SemaphoreType.DMA,
            pltpu.SemaphoreType.DMA,
        ],
        compiler_params=pltpu.CompilerParams(use_tc_tiling_on_sc=False),
    )
    def scat(hsf_hbm, row_hbm, col_hbm, zh_hbm, out_hbm,
             rowv, colv, rows0, rows1, acc, sg0, sg1):
        c = lax.axis_index("c")
        s = lax.axis_index("s")
        rpt = ROWS_PER_TILE
        pltpu.sync_copy(zh_hbm.at[pl.ds(s * rpt, rpt)], acc.at[pl.ds(s * rpt, rpt)])
        pltpu.sync_copy(row_hbm.at[c, s], rowv)
        pltpu.sync_copy(col_hbm.at[s], colv)
        plsc.subcore_barrier()

        pltpu.async_copy(hsf_hbm.at[rowv.at[0]], rows0, sg0)

        def body(t, _):
            j0 = 2 * t
            j1 = 2 * t + 1
            pltpu.make_async_copy(hsf_hbm.at[rowv.at[j0]], rows0, sg0).wait()
            pltpu.async_copy(hsf_hbm.at[rowv.at[j1]], rows1, sg1)
            pltpu.sync_copy(rows0, acc.at[colv.at[j0]], add=True)

            @pl.when(t + 1 < nch // 2)
            def _():
                pltpu.async_copy(hsf_hbm.at[rowv.at[j0 + 2]], rows0, sg0)

            pltpu.make_async_copy(hsf_hbm.at[rowv.at[j1]], rows1, sg1).wait()
            pltpu.sync_copy(rows1, acc.at[colv.at[j1]], add=True)
            return 0

        lax.fori_loop(0, nch // 2, body, 0)
        plsc.subcore_barrier()
        pltpu.sync_copy(acc.at[pl.ds(s * rpt, rpt)],
                        out_hbm.at[c, pl.ds(s * rpt, rpt)])

    return scat


def _dn_from_parts(d):
    dn = jnp.where(d > 0.0, lax.rsqrt(jnp.maximum(d, 1e-30)), 0.0)
    return dn[:, 0:1]


def _pre_body(x_ref, w_ref, b_ref, dp_ref, h_ref, hs_ref, gs_ref):
    pid = pl.program_id(0)
    h = jnp.maximum(jnp.dot(x_ref[...], w_ref[...],
                            preferred_element_type=jnp.float32) + b_ref[...], 0.0)
    r = pid * BN + lax.broadcasted_iota(jnp.int32, (BN, 1), 0)
    h = jnp.where(r < N, h, 0.0)
    dn = _dn_from_parts(dp_ref[...])
    h_ref[...] = h
    hs = h * dn
    hs_ref[0] = hs[:, :FH]
    hs_ref[1] = hs[:, FH:]

    @pl.when(pid == 0)
    def _():
        gs_ref[...] = jnp.zeros_like(gs_ref)

    gs_ref[...] += jnp.sum(h, axis=0, keepdims=True)


def _tc_pre(xp, w_in, b_in, dparts):
    return pl.pallas_call(
        _pre_body,
        grid=(NBLK,),
        in_specs=[
            pl.BlockSpec((BN, F), lambda b: (b, 0)),
            pl.BlockSpec((F, F), lambda b: (0, 0)),
            pl.BlockSpec((1, F), lambda b: (0, 0)),
            pl.BlockSpec((BN, 16), lambda b: (b, 0)),
        ],
        out_specs=[
            pl.BlockSpec((BN, F), lambda b: (b, 0)),
            pl.BlockSpec((2, BN, FH), lambda b: (0, b, 0)),
            pl.BlockSpec((1, F), lambda b: (0, 0)),
        ],
        out_shape=[
            jax.ShapeDtypeStruct((NPAD, F), jnp.float32),
            jax.ShapeDtypeStruct((2, NPAD, FH), jnp.float32),
            jax.ShapeDtypeStruct((1, F), jnp.float32),
        ],
    )(xp, w_in, b_in, dparts)


def _mix_body(last, acc_ref, dp_ref, h_ref, gs_ref, wl_ref, w1_ref, b1_ref,
              w2_ref, b2_ref, al_ref, fcw_ref, fcb_ref, cw_ref, wo_ref, bo_ref,
              *out_refs):
    pid = pl.program_id(0)
    dn = _dn_from_parts(dp_ref[...])
    acc = acc_ref[...]
    hi = dn * jnp.concatenate([acc[0], acc[1]], axis=1)
    h = h_ref[...]

    gp = gs_ref[...] * (1.0 / N)
    ge = jnp.maximum(jnp.dot(gp, w1_ref[...], preferred_element_type=jnp.float32)
                     + b1_ref[...], 0.0)
    ge = jnp.dot(ge, w2_ref[...], preferred_element_type=jnp.float32) + b2_ref[...]
    wgt = jax.nn.sigmoid(al_ref[...])

    local = jnp.dot(hi, wl_ref[...], preferred_element_type=jnp.float32)
    comb = wgt * local + (1.0 - wgt) * ge
    logits = jnp.dot(comb, fcw_ref[...], preferred_element_type=jnp.float32) + fcb_ref[...]
    m = jnp.max(logits, axis=1, keepdims=True)
    ee = jnp.exp(logits - m)
    e = ee / jnp.sum(ee, axis=1, keepdims=True)

    cw = cw_ref[...]
    out = h
    for k in range(K):
        ok = (jnp.dot(hi, cw[k, :F, :], preferred_element_type=jnp.float32)
              + jnp.dot(h, cw[k, F:, :], preferred_element_type=jnp.float32))
        out = out + e[:, k:k + 1] * ok
    hn = jnp.maximum(out, 0.0)

    if last:
        y_ref, = out_refs
        y_ref[...] = jnp.dot(hn, wo_ref[...], preferred_element_type=jnp.float32) + bo_ref[...]
    else:
        h_out, hs_out, gs_out = out_refs
        h_out[...] = hn
        hs = hn * dn
        hs_out[0] = hs[:, :FH]
        hs_out[1] = hs[:, FH:]

        @pl.when(pid == 0)
        def _():
            gs_out[...] = jnp.zeros_like(gs_out)

        gs_out[...] += jnp.sum(hn, axis=0, keepdims=True)


def _tc_mix(last, accs, dparts, h, gsum, wl, w1, b1, w2, b2, al, fcw, fcb, cw, wo, bo):
    if last:
        out_specs = [pl.BlockSpec((BN, F), lambda b: (b, 0))]
        out_shape = [jax.ShapeDtypeStruct((NPAD, F), jnp.float32)]
    else:
        out_specs = [
            pl.BlockSpec((BN, F), lambda b: (b, 0)),
            pl.BlockSpec((2, BN, FH), lambda b: (0, b, 0)),
            pl.BlockSpec((1, F), lambda b: (0, 0)),
        ]
        out_shape = [
            jax.ShapeDtypeStruct((NPAD, F), jnp.float32),
            jax.ShapeDtypeStruct((2, NPAD, FH), jnp.float32),
            jax.ShapeDtypeStruct((1, F), jnp.float32),
        ]
    return pl.pallas_call(
        functools.partial(_mix_body, last),
        grid=(NBLK,),
        in_specs=[
            pl.BlockSpec((2, BN, FH), lambda b: (0, b, 0)),
            pl.BlockSpec((BN, 16), lambda b: (b, 0)),
            pl.BlockSpec((BN, F), lambda b: (b, 0)),
            pl.BlockSpec((1, F), lambda b: (0, 0)),
            pl.BlockSpec((F, F), lambda b: (0, 0)),
            pl.BlockSpec((F, F), lambda b: (0, 0)),
            pl.BlockSpec((1, F), lambda b: (0, 0)),
            pl.BlockSpec((F, F), lambda b: (0, 0)),
            pl.BlockSpec((1, F), lambda b: (0, 0)),
            pl.BlockSpec((1, F), lambda b: (0, 0)),
            pl.BlockSpec((F, F), lambda b: (0, 0)),
            pl.BlockSpec((1, F), lambda b: (0, 0)),
            pl.BlockSpec((K, 2 * F, F), lambda b: (0, 0, 0)),
            pl.BlockSpec((F, F), lambda b: (0, 0)),
            pl.BlockSpec((1, F), lambda b: (0, 0)),
        ],
        out_specs=out_specs,
        out_shape=out_shape,
    )(accs, dparts, h, gsum, wl, w1, b1, w2, b2, al, fcw, fcb, cw, wo, bo)


def kernel(x, edge_index, W_in, b_in, conv_w, env_Wlocal, env_mlp_w1, env_mlp_b1,
           env_mlp_w2, env_mlp_b2, env_alpha, env_fc_w, env_fc_b, W_out, b_out):
    ei = edge_index.astype(jnp.int32)
    E = ei.shape[1]
    grp = 16 * CH * 2
    epad = ((E + grp - 1) // grp) * grp
    pad = epad - E
    rowp = jnp.concatenate([ei[0], jnp.full((pad,), N, jnp.int32)])
    colp = jnp.concatenate([ei[1], jnp.full((pad,), N, jnp.int32)])

    xp = jnp.zeros((NPAD, F), jnp.float32).at[:N].set(x)
    zh = jnp.zeros((NPAD, FH), jnp.float32)

    dparts = _sc_degree(epad)(colp)
    d16 = jnp.broadcast_to(jnp.sum(dparts, axis=0)[:, None], (NPAD, 16))

    b_in2 = b_in.reshape(1, F)
    h, hs2, gsum = _tc_pre(xp, W_in, b_in2, d16)

    fcw_p = jnp.zeros((L, F, F), jnp.float32).at[:, :, :K].set(env_fc_w)
    fcb_p = jnp.full((L, 1, F), -1e30, jnp.float32).at[:, 0, :K].set(env_fc_b)
    bo2 = b_out.reshape(1, F)

    nch = epad // (16 * CH)
    r16 = rowp.reshape(16, nch, CH)
    row2 = jnp.stack([r16, r16 + NPAD])
    col3 = colp.reshape(16, nch, CH)
    scat = _sc_scatter(epad)
    for l in range(L):
        accs = scat(hs2.reshape(2 * NPAD, FH), row2, col3, zh)
        al = jnp.full((1, F), env_alpha[l])
        outs = _tc_mix(l == L - 1, accs, d16, h, gsum,
                       env_Wlocal[l], env_mlp_w1[l], env_mlp_b1[l].reshape(1, F),
                       env_mlp_w2[l], env_mlp_b2[l].reshape(1, F), al,
                       fcw_p[l], fcb_p[l], conv_w[l], W_out, bo2)
        if l == L - 1:
            y, = outs
        else:
            h, hs2, gsum = outs
    return y[:N]
